# baseline (speedup 1.0000x reference)
import functools

import jax
import jax.numpy as jnp
from jax import lax
from jax.experimental import pallas as pl
from jax.experimental.pallas import tpu as pltpu
from jax.experimental.pallas import tpu_sc as plsc

N = 10000
E = 320000
H = 128
G = 16

NC = 2
NS = 16
NW = NC * NS
EP = E // NW
CH = 80
NJ = EP // CH
NP = 10240
RS = NP // NS

TE = 1280
GE = E // TE
TN = 1000
GN = N // TN
TBC = 8000

_f32 = jnp.float32


def _mesh():
    return plsc.VectorSubcoreMesh(core_axis_name="c", subcore_axis_name="s",
                                  num_cores=NC, num_subcores=NS)


@functools.lru_cache(maxsize=None)
def _get_sc_gather():
    @functools.partial(
        pl.kernel,
        out_type=jax.ShapeDtypeStruct((E, H), _f32),
        mesh=_mesh(),
        scratch_types=[
            pltpu.VMEM((NJ, CH), jnp.int32),
            pltpu.VMEM((NJ, CH), jnp.int32),
            pltpu.VMEM((CH, H), _f32),
            pltpu.VMEM((CH, H), _f32),
            pltpu.SemaphoreType.DMA,
            pltpu.SemaphoreType.DMA,
        ],
    )
    def k(xi_hbm, xj_hbm, dst3_hbm, src3_hbm, out_hbm,
          idxd, idxs, bufa, bufb, sema, semb):
        c = lax.axis_index("c")
        s = lax.axis_index("s")
        wid = s * NC + c
        base = wid * EP
        pltpu.sync_copy(dst3_hbm.at[wid], idxd)
        pltpu.sync_copy(src3_hbm.at[wid], idxs)

        def body(j, carry):
            a = pltpu.async_copy(xi_hbm.at[idxd.at[j]], bufa, sema)
            b = pltpu.async_copy(xj_hbm.at[idxs.at[j]], bufb, semb)
            a.wait()
            b.wait()
            for r in range(CH):
                for cc in range(H // 16):
                    sl = pl.ds(cc * 16, 16)
                    bufa[r, sl] = bufa[r, sl] + bufb[r, sl]
            pltpu.sync_copy(bufa, out_hbm.at[pl.ds(base + j * CH, CH)])
            return carry

        lax.fori_loop(0, NJ, body, 0)

    return k


def _sc_gather2(xi, xj, dst3, src3):
    return _get_sc_gather()(xi, xj, dst3, src3)


@functools.lru_cache(maxsize=None)
def _get_sc_scatter():
    @functools.partial(
        pl.kernel,
        out_type=jax.ShapeDtypeStruct((NC, NP, H), _f32),
        mesh=_mesh(),
        scratch_types=[
            pltpu.VMEM((NJ, CH), jnp.int32),
            pltpu.VMEM((CH, H), _f32),
            pltpu.VMEM_SHARED((NP, H), _f32),
        ],
    )
    def k(vals_hbm, idx3_hbm, zeros_hbm, out_hbm, idxv, chunk, acc):
        c = lax.axis_index("c")
        s = lax.axis_index("s")
        wid = s * NC + c
        base = wid * EP
        pltpu.sync_copy(idx3_hbm.at[wid], idxv)
        pltpu.sync_copy(zeros_hbm.at[pl.ds(s * RS, RS)],
                        acc.at[pl.ds(s * RS, RS)])
        plsc.subcore_barrier()

        def body(j, carry):
            pltpu.sync_copy(vals_hbm.at[pl.ds(base + j * CH, CH)], chunk)
            pltpu.sync_copy(chunk, acc.at[idxv.at[j]], add=True)
            return carry

        lax.fori_loop(0, NJ, body, 0)
        plsc.subcore_barrier()
        pltpu.sync_copy(acc.at[pl.ds(s * RS, RS)],
                        out_hbm.at[c, pl.ds(s * RS, RS)])

    return k


def _sc_scatter(vals, idx3, zeros):
    return _get_sc_scatter()(vals, idx3, zeros)


@functools.lru_cache(maxsize=None)
def _get_sc_hist():
    @functools.partial(
        pl.kernel,
        out_type=jax.ShapeDtypeStruct((NW, 8, H), _f32),
        mesh=_mesh(),
        scratch_types=[
            pltpu.VMEM((NJ, CH), jnp.int32),
            pltpu.VMEM((CH, H), _f32),
            pltpu.VMEM((8, H), _f32),
            pltpu.SemaphoreType.DMA,
        ],
    )
    def k(tab_hbm, src3_hbm, zeros8_hbm, out_hbm, idxs, buf, accv, sem):
        c = lax.axis_index("c")
        s = lax.axis_index("s")
        wid = s * NC + c
        pltpu.sync_copy(src3_hbm.at[wid], idxs)
        pltpu.sync_copy(zeros8_hbm, accv)

        def body(j, acc):
            pltpu.async_copy(tab_hbm.at[idxs.at[j]], buf, sem).wait()
            for r in range(CH):
                acc = acc + buf[r, pl.ds(0, 16)]
            return acc

        acc = lax.fori_loop(0, NJ, body, jnp.zeros((16,), _f32))
        accv[0, pl.ds(0, 16)] = acc
        pltpu.sync_copy(accv, out_hbm.at[wid])

    return k


def _sc_hist(onehot128, src3, zeros8):
    return _get_sc_hist()(onehot128, src3, zeros8)


def _dot(a, b):
    return jnp.dot(a, b, preferred_element_type=_f32)


def _wspec(shape):
    return pl.BlockSpec(shape, lambda i: (0,) * len(shape))


def _tc_node0(xpad, w1, w2, w3, bmat, wi, wj):
    def body(x_ref, w1_ref, w2_ref, w3_ref, b_ref, wi_ref, wj_ref,
             x0_ref, xi_ref, xj_ref):
        h = jnp.maximum(x_ref[:, 0:1] * w1_ref[0:1, :] + b_ref[0], 0.0)
        h = jnp.maximum(_dot(h, w2_ref[...]) + b_ref[1], 0.0)
        x0 = _dot(h, w3_ref[...]) + b_ref[2]
        x0_ref[...] = x0
        xi_ref[...] = _dot(x0, wi_ref[...])
        xj_ref[...] = _dot(x0, wj_ref[...])

    return pl.pallas_call(
        body,
        grid=(GN,),
        in_specs=[
            pl.BlockSpec((TN, 8), lambda i: (i, 0)),
            _wspec((8, H)), _wspec((H, H)), _wspec((H, H)),
            _wspec((8, H)), _wspec((H, H)), _wspec((H, H)),
        ],
        out_specs=[pl.BlockSpec((TN, H), lambda i: (i, 0))] * 3,
        out_shape=[jax.ShapeDtypeStruct((N, H), _f32)] * 3,
    )(xpad, w1, w2, w3, bmat, wi, wj)


def _tc_edge1(attr8, gpre, wa1, wa2, wa3, we1ef, we2, we3, gfe, starts8,
              ends8, bmat):
    def body(a_ref, g_ref, wa1_ref, wa2_ref, wa3_ref, wef_ref,
             we2_ref, we3_ref, gfe_ref, st_ref, en_ref, b_ref,
             ef0_ref, msg_ref):
        i = pl.program_id(0)
        a = a_ref[...]
        h = jnp.maximum(_dot(a, wa1_ref[...]) + b_ref[0], 0.0)
        h = jnp.maximum(_dot(h, wa2_ref[...]) + b_ref[1], 0.0)
        ef0 = _dot(h, wa3_ref[...]) + b_ref[2]
        ef0_ref[...] = ef0
        pos = (lax.broadcasted_iota(jnp.int32, (TE, 1), 0)
               + i * TE).astype(_f32)
        st = st_ref[0:1, 0:G]
        en = en_ref[0:1, 0:G]
        oh = ((pos >= st) & (pos < en)).astype(_f32)
        gfm = _dot(oh, gfe_ref[...])
        h1 = jnp.maximum(_dot(ef0, wef_ref[...]) + g_ref[...]
                         + gfm + b_ref[3], 0.0)
        h2 = jnp.maximum(_dot(h1, we2_ref[...]) + b_ref[4], 0.0)
        msg_ref[...] = _dot(h2, we3_ref[...]) + b_ref[5]

    return pl.pallas_call(
        body,
        grid=(GE,),
        in_specs=[
            pl.BlockSpec((TE, 8), lambda i: (i, 0)),
            pl.BlockSpec((TE, H), lambda i: (i, 0)),
            _wspec((8, H)), _wspec((H, H)), _wspec((H, H)),
            _wspec((H, H)), _wspec((H, H)), _wspec((H, H)),
            _wspec((G, H)), _wspec((8, 32)), _wspec((8, 32)),
            _wspec((8, H)),
        ],
        out_specs=[pl.BlockSpec((TE, H), lambda i: (i, 0))] * 2,
        out_shape=[jax.ShapeDtypeStruct((E, H), _f32)] * 2,
    )(attr8, gpre, wa1, wa2, wa3, we1ef, we2, we3, gfe, starts8, ends8,
      bmat)


def _tc_node1(x0, seg, onehotn, wx, wa, gfn, w2, w3, bmat, wi, wj):
    def body(x_ref, seg_ref, oh_ref, wx_ref, wa_ref, gfn_ref, w2_ref,
             w3_ref, b_ref, wi_ref, wj_ref, x1_ref, xi_ref, xj_ref):
        x = x_ref[...]
        aggr = seg_ref[0] + seg_ref[1]
        gfm = _dot(oh_ref[...], gfn_ref[...])
        h = jnp.maximum(_dot(x, wx_ref[...]) + _dot(aggr, wa_ref[...])
                        + gfm + b_ref[0], 0.0)
        h = jnp.maximum(_dot(h, w2_ref[...]) + b_ref[1], 0.0)
        x1 = x + _dot(h, w3_ref[...]) + b_ref[2]
        x1_ref[...] = x1
        xi_ref[...] = _dot(x1, wi_ref[...])
        xj_ref[...] = _dot(x1, wj_ref[...])

    return pl.pallas_call(
        body,
        grid=(GN,),
        in_specs=[
            pl.BlockSpec((TN, H), lambda i: (i, 0)),
            pl.BlockSpec((NC, TN, H), lambda i: (0, i, 0)),
            pl.BlockSpec((TN, G), lambda i: (i, 0)),
            _wspec((H, H)), _wspec((H, H)), _wspec((G, H)),
            _wspec((H, H)), _wspec((H, H)), _wspec((8, H)),
            _wspec((H, H)), _wspec((H, H)),
        ],
        out_specs=[pl.BlockSpec((TN, H), lambda i: (i, 0))] * 3,
        out_shape=[jax.ShapeDtypeStruct((N, H), _f32)] * 3,
    )(x0, seg, onehotn, wx, wa, gfn, w2, w3, bmat, wi, wj)


def _tc_edge2(ef0, msg1, gpre, we1ef, we2, we3, gfe, starts8, ends8, bmat):
    def body(ef0_ref, m1_ref, g_ref, wef_ref, we2_ref, we3_ref,
             gfe_ref, st_ref, en_ref, b_ref, msg_ref):
        i = pl.program_id(0)
        ef1 = ef0_ref[...] + m1_ref[...]
        pos = (lax.broadcasted_iota(jnp.int32, (TE, 1), 0)
               + i * TE).astype(_f32)
        st = st_ref[0:1, 0:G]
        en = en_ref[0:1, 0:G]
        oh = ((pos >= st) & (pos < en)).astype(_f32)
        gfm = _dot(oh, gfe_ref[...])
        h1 = jnp.maximum(_dot(ef1, wef_ref[...]) + g_ref[...]
                         + gfm + b_ref[0], 0.0)
        h2 = jnp.maximum(_dot(h1, we2_ref[...]) + b_ref[1], 0.0)
        msg_ref[...] = _dot(h2, we3_ref[...]) + b_ref[2]

    return pl.pallas_call(
        body,
        grid=(GE,),
        in_specs=[
            pl.BlockSpec((TE, H), lambda i: (i, 0)),
            pl.BlockSpec((TE, H), lambda i: (i, 0)),
            pl.BlockSpec((TE, H), lambda i: (i, 0)),
            _wspec((H, H)), _wspec((H, H)), _wspec((H, H)),
            _wspec((G, H)), _wspec((8, 32)), _wspec((8, 32)),
            _wspec((8, H)),
        ],
        out_specs=pl.BlockSpec((TE, H), lambda i: (i, 0)),
        out_shape=jax.ShapeDtypeStruct((E, H), _f32),
    )(ef0, msg1, gpre, we1ef, we2, we3, gfe, starts8, ends8, bmat)


def _tc_node2(x1, seg, onehotn, wx, wa, gfn, w2, w3, bmat, wact1, wact2,
              wact3p, bact):
    def body(x_ref, seg_ref, oh_ref, wx_ref, wa_ref, gfn_ref, w2_ref,
             w3_ref, b_ref, a1_ref, a2_ref, a3_ref, ba_ref, out_ref):
        x = x_ref[...]
        aggr = seg_ref[0] + seg_ref[1]
        gfm = _dot(oh_ref[...], gfn_ref[...])
        h = jnp.maximum(_dot(x, wx_ref[...]) + _dot(aggr, wa_ref[...])
                        + gfm + b_ref[0], 0.0)
        h = jnp.maximum(_dot(h, w2_ref[...]) + b_ref[1], 0.0)
        x2 = x + _dot(h, w3_ref[...]) + b_ref[2]
        a = jnp.maximum(_dot(x2, a1_ref[...]) + ba_ref[0], 0.0)
        a = jnp.maximum(_dot(a, a2_ref[...]) + ba_ref[1], 0.0)
        out_ref[...] = _dot(a, a3_ref[...]) + ba_ref[2]

    return pl.pallas_call(
        body,
        grid=(GN,),
        in_specs=[
            pl.BlockSpec((TN, H), lambda i: (i, 0)),
            pl.BlockSpec((NC, TN, H), lambda i: (0, i, 0)),
            pl.BlockSpec((TN, G), lambda i: (i, 0)),
            _wspec((H, H)), _wspec((H, H)), _wspec((G, H)),
            _wspec((H, H)), _wspec((H, H)), _wspec((8, H)),
            _wspec((H, H)), _wspec((H, H)), _wspec((H, H)),
            _wspec((8, H)),
        ],
        out_specs=pl.BlockSpec((TN, H), lambda i: (i, 0)),
        out_shape=jax.ShapeDtypeStruct((N, H), _f32),
    )(x1, seg, onehotn, wx, wa, gfn, w2, w3, bmat, wact1, wact2, wact3p,
      bact)


def _bias_mat(*bs):
    m = jnp.zeros((8, H), _f32)
    for r, b in enumerate(bs):
        m = m.at[r, : b.shape[0]].set(b)
    return m


def _pad_rows(w, rows):
    return jnp.zeros((rows, w.shape[1]), _f32).at[: w.shape[0]].set(w)


def kernel(contact_node, contact_force, initial_edge_delta, length,
           parent2child, branch, stiffness, edge_index, batch, params):
    src = edge_index[0].astype(jnp.int32)
    dst = edge_index[1].astype(jnp.int32)
    dst3 = dst.reshape(NW, NJ, CH)
    src3 = src.reshape(NW, NJ, CH)

    attr8 = jnp.concatenate(
        [initial_edge_delta, length[:, None], parent2child[:, None],
         branch[:, None], stiffness[:, None], jnp.zeros((E, 1), _f32)],
        axis=1)
    xpad = jnp.pad(contact_node[:, None].astype(_f32), ((0, 0), (0, 7)))
    onehotn = (batch[:, None] == jnp.arange(G, dtype=batch.dtype)[None, :]
               ).astype(_f32)
    zerosH = jnp.zeros((NP, H), _f32)
    gf = contact_force.reshape(G, 3)

    (wn01, bn01), (wn02, bn02), (wn03, bn03) = params['node_in']
    (wa1, ba1), (wa2, ba2), (wa3, ba3) = params['edge_in']
    (wc1, bc1), (wc2, bc2), (wc3, bc3) = params['node_action']
    in1, in2 = params['IN'][0], params['IN'][1]
    (e11, be11), (e12, be12), (e13, be13) = in1['lin_edge']
    (n11, bn11), (n12, bn12), (n13, bn13) = in1['lin_node']
    (e21, be21), (e22, be22), (e23, be23) = in2['lin_edge']
    (n21, bn21), (n22, bn22), (n23, bn23) = in2['lin_node']

    w1i_1, w1j_1, w1e_1, w1g_1 = e11[:H], e11[H:2*H], e11[2*H:3*H], e11[3*H:]
    w1i_2, w1j_2, w1e_2, w1g_2 = e21[:H], e21[H:2*H], e21[2*H:3*H], e21[3*H:]
    nx1, na1, ng1 = n11[:H], n11[H:2*H], n11[2*H:]
    nx2, na2, ng2 = n21[:H], n21[H:2*H], n21[2*H:]

    gfe1 = _dot(gf, w1g_1)
    gfe2 = _dot(gf, w1g_2)
    gfn1 = _dot(gf, ng1)
    gfn2 = _dot(gf, ng2)

    onehot128 = jnp.pad(onehotn, ((0, 0), (0, H - G)))
    hist = _sc_hist(onehot128, src3, jnp.zeros((8, H), _f32))
    counts = jnp.sum(hist, axis=(0, 1))[:G]
    ends_v = jnp.cumsum(counts)
    starts_v = ends_v - counts
    starts8 = jnp.broadcast_to(
        jnp.zeros((32,), _f32).at[:G].set(starts_v), (8, 32))
    ends8 = jnp.broadcast_to(
        jnp.zeros((32,), _f32).at[:G].set(ends_v), (8, 32))

    x0, xi1, xj1 = _tc_node0(
        xpad, _pad_rows(wn01, 8), wn02, wn03,
        _bias_mat(bn01, bn02, bn03), w1i_1, w1j_1)

    gpre1 = _sc_gather2(xi1, xj1, dst3, src3)
    ef0, msg1 = _tc_edge1(
        attr8, gpre1, _pad_rows(wa1, 8), wa2, wa3, w1e_1, e12, e13,
        gfe1, starts8, ends8, _bias_mat(ba1, ba2, ba3, be11, be12, be13))
    seg1 = _sc_scatter(msg1, dst3, zerosH)
    x1, xi2, xj2 = _tc_node1(
        x0, seg1, onehotn, nx1, na1, gfn1, n12, n13,
        _bias_mat(bn11, bn12, bn13), w1i_2, w1j_2)

    gpre2 = _sc_gather2(xi2, xj2, dst3, src3)
    msg2 = _tc_edge2(
        ef0, msg1, gpre2, w1e_2, e22, e23, gfe2, starts8, ends8,
        _bias_mat(be21, be22, be23))
    seg2 = _sc_scatter(msg2, dst3, zerosH)

    wc3p = jnp.zeros((H, H), _f32).at[:, :3].set(wc3)
    bc3p = jnp.zeros((H,), _f32).at[:3].set(bc3)
    out = _tc_node2(
        x1, seg2, onehotn, nx2, na2, gfn2, n22, n23,
        _bias_mat(bn21, bn22, bn23), wc1, wc2, wc3p,
        _bias_mat(bc1, bc2, bc3p))
    return out[:, :3]


# speedup vs baseline: 1.2409x; 1.2409x over previous
import functools

import jax
import jax.numpy as jnp
from jax import lax
from jax.experimental import pallas as pl
from jax.experimental.pallas import tpu as pltpu
from jax.experimental.pallas import tpu_sc as plsc

N = 10000
E = 320000
H = 128
G = 16

NC = 2
NS = 16
NW = NC * NS
EP = E // NW
CH = 80
NJ = EP // CH
NP = 10240
RS = NP // NS

TE = 1280
GE = E // TE
TN = 1000
GN = N // TN
TBC = 8000

_f32 = jnp.float32


def _mesh():
    return plsc.VectorSubcoreMesh(core_axis_name="c", subcore_axis_name="s",
                                  num_cores=NC, num_subcores=NS)


@functools.lru_cache(maxsize=None)
def _get_sc_gather():
    @functools.partial(
        pl.kernel,
        out_type=(jax.ShapeDtypeStruct((E, H), _f32),
                  jax.ShapeDtypeStruct((E, H), _f32)),
        mesh=_mesh(),
        scratch_types=[
            pltpu.VMEM((NJ, CH), jnp.int32),
            pltpu.VMEM((NJ, CH), jnp.int32),
            pltpu.VMEM((CH, H), _f32),
            pltpu.VMEM((CH, H), _f32),
            pltpu.SemaphoreType.DMA,
            pltpu.SemaphoreType.DMA,
        ],
    )
    def k(xi_hbm, xj_hbm, dst3_hbm, src3_hbm, outa_hbm, outb_hbm,
          idxd, idxs, bufa, bufb, sema, semb):
        c = lax.axis_index("c")
        s = lax.axis_index("s")
        wid = s * NC + c
        base = wid * EP
        pltpu.sync_copy(dst3_hbm.at[wid], idxd)
        pltpu.sync_copy(src3_hbm.at[wid], idxs)

        def body(j, carry):
            a = pltpu.async_copy(xi_hbm.at[idxd.at[j]], bufa, sema)
            b = pltpu.async_copy(xj_hbm.at[idxs.at[j]], bufb, semb)
            a.wait()
            b.wait()
            pltpu.sync_copy(bufa, outa_hbm.at[pl.ds(base + j * CH, CH)])
            pltpu.sync_copy(bufb, outb_hbm.at[pl.ds(base + j * CH, CH)])
            return carry

        lax.fori_loop(0, NJ, body, 0)

    return k


def _sc_gather2(xi, xj, dst3, src3):
    return _get_sc_gather()(xi, xj, dst3, src3)


@functools.lru_cache(maxsize=None)
def _get_sc_scatter():
    @functools.partial(
        pl.kernel,
        out_type=jax.ShapeDtypeStruct((NC, NP, H), _f32),
        mesh=_mesh(),
        scratch_types=[
            pltpu.VMEM((NJ, CH), jnp.int32),
            pltpu.VMEM((CH, H), _f32),
            pltpu.VMEM_SHARED((NP, H), _f32),
        ],
    )
    def k(vals_hbm, idx3_hbm, zeros_hbm, out_hbm, idxv, chunk, acc):
        c = lax.axis_index("c")
        s = lax.axis_index("s")
        wid = s * NC + c
        base = wid * EP
        pltpu.sync_copy(idx3_hbm.at[wid], idxv)
        pltpu.sync_copy(zeros_hbm.at[pl.ds(s * RS, RS)],
                        acc.at[pl.ds(s * RS, RS)])
        plsc.subcore_barrier()

        def body(j, carry):
            pltpu.sync_copy(vals_hbm.at[pl.ds(base + j * CH, CH)], chunk)
            pltpu.sync_copy(chunk, acc.at[idxv.at[j]], add=True)
            return carry

        lax.fori_loop(0, NJ, body, 0)
        plsc.subcore_barrier()
        pltpu.sync_copy(acc.at[pl.ds(s * RS, RS)],
                        out_hbm.at[c, pl.ds(s * RS, RS)])

    return k


def _sc_scatter(vals, idx3, zeros):
    return _get_sc_scatter()(vals, idx3, zeros)


@functools.lru_cache(maxsize=None)
def _get_sc_hist():
    @functools.partial(
        pl.kernel,
        out_type=jax.ShapeDtypeStruct((NW, 8, H), _f32),
        mesh=_mesh(),
        scratch_types=[
            pltpu.VMEM((NJ, CH), jnp.int32),
            pltpu.VMEM((CH, H), _f32),
            pltpu.VMEM((8, H), _f32),
            pltpu.SemaphoreType.DMA,
        ],
    )
    def k(tab_hbm, src3_hbm, zeros8_hbm, out_hbm, idxs, buf, accv, sem):
        c = lax.axis_index("c")
        s = lax.axis_index("s")
        wid = s * NC + c
        pltpu.sync_copy(src3_hbm.at[wid], idxs)
        pltpu.sync_copy(zeros8_hbm, accv)

        def body(j, acc):
            pltpu.async_copy(tab_hbm.at[idxs.at[j]], buf, sem).wait()
            for r in range(CH):
                acc = acc + buf[r, pl.ds(0, 16)]
            return acc

        acc = lax.fori_loop(0, NJ, body, jnp.zeros((16,), _f32))
        accv[0, pl.ds(0, 16)] = acc
        pltpu.sync_copy(accv, out_hbm.at[wid])

    return k


def _sc_hist(onehot128, src3, zeros8):
    return _get_sc_hist()(onehot128, src3, zeros8)


def _dot(a, b):
    return jnp.dot(a, b, preferred_element_type=_f32)


def _wspec(shape):
    return pl.BlockSpec(shape, lambda i: (0,) * len(shape))


def _tc_node0(xpad, w1, w2, w3, bmat, wi, wj):
    def body(x_ref, w1_ref, w2_ref, w3_ref, b_ref, wi_ref, wj_ref,
             x0_ref, xi_ref, xj_ref):
        h = jnp.maximum(x_ref[:, 0:1] * w1_ref[0:1, :] + b_ref[0], 0.0)
        h = jnp.maximum(_dot(h, w2_ref[...]) + b_ref[1], 0.0)
        x0 = _dot(h, w3_ref[...]) + b_ref[2]
        x0_ref[...] = x0
        xi_ref[...] = _dot(x0, wi_ref[...])
        xj_ref[...] = _dot(x0, wj_ref[...])

    return pl.pallas_call(
        body,
        grid=(GN,),
        in_specs=[
            pl.BlockSpec((TN, 8), lambda i: (i, 0)),
            _wspec((8, H)), _wspec((H, H)), _wspec((H, H)),
            _wspec((8, H)), _wspec((H, H)), _wspec((H, H)),
        ],
        out_specs=[pl.BlockSpec((TN, H), lambda i: (i, 0))] * 3,
        out_shape=[jax.ShapeDtypeStruct((N, H), _f32)] * 3,
    )(xpad, w1, w2, w3, bmat, wi, wj)


def _tc_edge1(attr8, ga, gb, wa1, wa2, wa3, we1ef, we2, we3, gfe, starts8,
              ends8, bmat):
    def body(a_ref, ga_ref, gb_ref, wa1_ref, wa2_ref, wa3_ref, wef_ref,
             we2_ref, we3_ref, gfe_ref, st_ref, en_ref, b_ref,
             ef0_ref, msg_ref):
        i = pl.program_id(0)
        a = a_ref[...]
        h = jnp.maximum(_dot(a, wa1_ref[...]) + b_ref[0], 0.0)
        h = jnp.maximum(_dot(h, wa2_ref[...]) + b_ref[1], 0.0)
        ef0 = _dot(h, wa3_ref[...]) + b_ref[2]
        ef0_ref[...] = ef0
        pos = (lax.broadcasted_iota(jnp.int32, (TE, 1), 0)
               + i * TE).astype(_f32)
        st = st_ref[0:1, 0:G]
        en = en_ref[0:1, 0:G]
        oh = ((pos >= st) & (pos < en)).astype(_f32)
        gfm = _dot(oh, gfe_ref[...])
        h1 = jnp.maximum(_dot(ef0, wef_ref[...]) + (ga_ref[...] + gb_ref[...])
                         + gfm + b_ref[3], 0.0)
        h2 = jnp.maximum(_dot(h1, we2_ref[...]) + b_ref[4], 0.0)
        msg_ref[...] = _dot(h2, we3_ref[...]) + b_ref[5]

    return pl.pallas_call(
        body,
        grid=(GE,),
        in_specs=[
            pl.BlockSpec((TE, 8), lambda i: (i, 0)),
            pl.BlockSpec((TE, H), lambda i: (i, 0)),
            pl.BlockSpec((TE, H), lambda i: (i, 0)),
            _wspec((8, H)), _wspec((H, H)), _wspec((H, H)),
            _wspec((H, H)), _wspec((H, H)), _wspec((H, H)),
            _wspec((G, H)), _wspec((8, 32)), _wspec((8, 32)),
            _wspec((8, H)),
        ],
        out_specs=[pl.BlockSpec((TE, H), lambda i: (i, 0))] * 2,
        out_shape=[jax.ShapeDtypeStruct((E, H), _f32)] * 2,
    )(attr8, ga, gb, wa1, wa2, wa3, we1ef, we2, we3, gfe, starts8, ends8,
      bmat)


def _tc_node1(x0, seg, onehotn, wx, wa, gfn, w2, w3, bmat, wi, wj):
    def body(x_ref, seg_ref, oh_ref, wx_ref, wa_ref, gfn_ref, w2_ref,
             w3_ref, b_ref, wi_ref, wj_ref, x1_ref, xi_ref, xj_ref):
        x = x_ref[...]
        aggr = seg_ref[0] + seg_ref[1]
        gfm = _dot(oh_ref[...], gfn_ref[...])
        h = jnp.maximum(_dot(x, wx_ref[...]) + _dot(aggr, wa_ref[...])
                        + gfm + b_ref[0], 0.0)
        h = jnp.maximum(_dot(h, w2_ref[...]) + b_ref[1], 0.0)
        x1 = x + _dot(h, w3_ref[...]) + b_ref[2]
        x1_ref[...] = x1
        xi_ref[...] = _dot(x1, wi_ref[...])
        xj_ref[...] = _dot(x1, wj_ref[...])

    return pl.pallas_call(
        body,
        grid=(GN,),
        in_specs=[
            pl.BlockSpec((TN, H), lambda i: (i, 0)),
            pl.BlockSpec((NC, TN, H), lambda i: (0, i, 0)),
            pl.BlockSpec((TN, G), lambda i: (i, 0)),
            _wspec((H, H)), _wspec((H, H)), _wspec((G, H)),
            _wspec((H, H)), _wspec((H, H)), _wspec((8, H)),
            _wspec((H, H)), _wspec((H, H)),
        ],
        out_specs=[pl.BlockSpec((TN, H), lambda i: (i, 0))] * 3,
        out_shape=[jax.ShapeDtypeStruct((N, H), _f32)] * 3,
    )(x0, seg, onehotn, wx, wa, gfn, w2, w3, bmat, wi, wj)


def _tc_edge2(ef0, msg1, ga, gb, we1ef, we2, we3, gfe, starts8, ends8, bmat):
    def body(ef0_ref, m1_ref, ga_ref, gb_ref, wef_ref, we2_ref, we3_ref,
             gfe_ref, st_ref, en_ref, b_ref, msg_ref):
        i = pl.program_id(0)
        ef1 = ef0_ref[...] + m1_ref[...]
        pos = (lax.broadcasted_iota(jnp.int32, (TE, 1), 0)
               + i * TE).astype(_f32)
        st = st_ref[0:1, 0:G]
        en = en_ref[0:1, 0:G]
        oh = ((pos >= st) & (pos < en)).astype(_f32)
        gfm = _dot(oh, gfe_ref[...])
        h1 = jnp.maximum(_dot(ef1, wef_ref[...]) + (ga_ref[...] + gb_ref[...])
                         + gfm + b_ref[0], 0.0)
        h2 = jnp.maximum(_dot(h1, we2_ref[...]) + b_ref[1], 0.0)
        msg_ref[...] = _dot(h2, we3_ref[...]) + b_ref[2]

    return pl.pallas_call(
        body,
        grid=(GE,),
        in_specs=[
            pl.BlockSpec((TE, H), lambda i: (i, 0)),
            pl.BlockSpec((TE, H), lambda i: (i, 0)),
            pl.BlockSpec((TE, H), lambda i: (i, 0)),
            pl.BlockSpec((TE, H), lambda i: (i, 0)),
            _wspec((H, H)), _wspec((H, H)), _wspec((H, H)),
            _wspec((G, H)), _wspec((8, 32)), _wspec((8, 32)),
            _wspec((8, H)),
        ],
        out_specs=pl.BlockSpec((TE, H), lambda i: (i, 0)),
        out_shape=jax.ShapeDtypeStruct((E, H), _f32),
    )(ef0, msg1, ga, gb, we1ef, we2, we3, gfe, starts8, ends8, bmat)


def _tc_node2(x1, seg, onehotn, wx, wa, gfn, w2, w3, bmat, wact1, wact2,
              wact3p, bact):
    def body(x_ref, seg_ref, oh_ref, wx_ref, wa_ref, gfn_ref, w2_ref,
             w3_ref, b_ref, a1_ref, a2_ref, a3_ref, ba_ref, out_ref):
        x = x_ref[...]
        aggr = seg_ref[0] + seg_ref[1]
        gfm = _dot(oh_ref[...], gfn_ref[...])
        h = jnp.maximum(_dot(x, wx_ref[...]) + _dot(aggr, wa_ref[...])
                        + gfm + b_ref[0], 0.0)
        h = jnp.maximum(_dot(h, w2_ref[...]) + b_ref[1], 0.0)
        x2 = x + _dot(h, w3_ref[...]) + b_ref[2]
        a = jnp.maximum(_dot(x2, a1_ref[...]) + ba_ref[0], 0.0)
        a = jnp.maximum(_dot(a, a2_ref[...]) + ba_ref[1], 0.0)
        out_ref[...] = _dot(a, a3_ref[...]) + ba_ref[2]

    return pl.pallas_call(
        body,
        grid=(GN,),
        in_specs=[
            pl.BlockSpec((TN, H), lambda i: (i, 0)),
            pl.BlockSpec((NC, TN, H), lambda i: (0, i, 0)),
            pl.BlockSpec((TN, G), lambda i: (i, 0)),
            _wspec((H, H)), _wspec((H, H)), _wspec((G, H)),
            _wspec((H, H)), _wspec((H, H)), _wspec((8, H)),
            _wspec((H, H)), _wspec((H, H)), _wspec((H, H)),
            _wspec((8, H)),
        ],
        out_specs=pl.BlockSpec((TN, H), lambda i: (i, 0)),
        out_shape=jax.ShapeDtypeStruct((N, H), _f32),
    )(x1, seg, onehotn, wx, wa, gfn, w2, w3, bmat, wact1, wact2, wact3p,
      bact)


def _bias_mat(*bs):
    m = jnp.zeros((8, H), _f32)
    for r, b in enumerate(bs):
        m = m.at[r, : b.shape[0]].set(b)
    return m


def _pad_rows(w, rows):
    return jnp.zeros((rows, w.shape[1]), _f32).at[: w.shape[0]].set(w)


def kernel(contact_node, contact_force, initial_edge_delta, length,
           parent2child, branch, stiffness, edge_index, batch, params):
    src = edge_index[0].astype(jnp.int32)
    dst = edge_index[1].astype(jnp.int32)
    dst3 = dst.reshape(NW, NJ, CH)
    src3 = src.reshape(NW, NJ, CH)

    attr8 = jnp.concatenate(
        [initial_edge_delta, length[:, None], parent2child[:, None],
         branch[:, None], stiffness[:, None], jnp.zeros((E, 1), _f32)],
        axis=1)
    xpad = jnp.pad(contact_node[:, None].astype(_f32), ((0, 0), (0, 7)))
    onehotn = (batch[:, None] == jnp.arange(G, dtype=batch.dtype)[None, :]
               ).astype(_f32)
    zerosH = jnp.zeros((NP, H), _f32)
    gf = contact_force.reshape(G, 3)

    (wn01, bn01), (wn02, bn02), (wn03, bn03) = params['node_in']
    (wa1, ba1), (wa2, ba2), (wa3, ba3) = params['edge_in']
    (wc1, bc1), (wc2, bc2), (wc3, bc3) = params['node_action']
    in1, in2 = params['IN'][0], params['IN'][1]
    (e11, be11), (e12, be12), (e13, be13) = in1['lin_edge']
    (n11, bn11), (n12, bn12), (n13, bn13) = in1['lin_node']
    (e21, be21), (e22, be22), (e23, be23) = in2['lin_edge']
    (n21, bn21), (n22, bn22), (n23, bn23) = in2['lin_node']

    w1i_1, w1j_1, w1e_1, w1g_1 = e11[:H], e11[H:2*H], e11[2*H:3*H], e11[3*H:]
    w1i_2, w1j_2, w1e_2, w1g_2 = e21[:H], e21[H:2*H], e21[2*H:3*H], e21[3*H:]
    nx1, na1, ng1 = n11[:H], n11[H:2*H], n11[2*H:]
    nx2, na2, ng2 = n21[:H], n21[H:2*H], n21[2*H:]

    gfe1 = _dot(gf, w1g_1)
    gfe2 = _dot(gf, w1g_2)
    gfn1 = _dot(gf, ng1)
    gfn2 = _dot(gf, ng2)

    onehot128 = jnp.pad(onehotn, ((0, 0), (0, H - G)))
    hist = _sc_hist(onehot128, src3, jnp.zeros((8, H), _f32))
    counts = jnp.sum(hist, axis=(0, 1))[:G]
    ends_v = jnp.cumsum(counts)
    starts_v = ends_v - counts
    starts8 = jnp.broadcast_to(
        jnp.zeros((32,), _f32).at[:G].set(starts_v), (8, 32))
    ends8 = jnp.broadcast_to(
        jnp.zeros((32,), _f32).at[:G].set(ends_v), (8, 32))

    x0, xi1, xj1 = _tc_node0(
        xpad, _pad_rows(wn01, 8), wn02, wn03,
        _bias_mat(bn01, bn02, bn03), w1i_1, w1j_1)

    ga1, gb1 = _sc_gather2(xi1, xj1, dst3, src3)
    ef0, msg1 = _tc_edge1(
        attr8, ga1, gb1, _pad_rows(wa1, 8), wa2, wa3, w1e_1, e12, e13,
        gfe1, starts8, ends8, _bias_mat(ba1, ba2, ba3, be11, be12, be13))
    seg1 = _sc_scatter(msg1, dst3, zerosH)
    x1, xi2, xj2 = _tc_node1(
        x0, seg1, onehotn, nx1, na1, gfn1, n12, n13,
        _bias_mat(bn11, bn12, bn13), w1i_2, w1j_2)

    ga2, gb2 = _sc_gather2(xi2, xj2, dst3, src3)
    msg2 = _tc_edge2(
        ef0, msg1, ga2, gb2, w1e_2, e22, e23, gfe2, starts8, ends8,
        _bias_mat(be21, be22, be23))
    seg2 = _sc_scatter(msg2, dst3, zerosH)

    wc3p = jnp.zeros((H, H), _f32).at[:, :3].set(wc3)
    bc3p = jnp.zeros((H,), _f32).at[:3].set(bc3)
    out = _tc_node2(
        x1, seg2, onehotn, nx2, na2, gfn2, n22, n23,
        _bias_mat(bn21, bn22, bn23), wc1, wc2, wc3p,
        _bias_mat(bc1, bc2, bc3p))
    return out[:, :3]


# speedup vs baseline: 1.2664x; 1.0205x over previous
import functools

import jax
import jax.numpy as jnp
from jax import lax
from jax.experimental import pallas as pl
from jax.experimental.pallas import tpu as pltpu
from jax.experimental.pallas import tpu_sc as plsc

N = 10000
E = 320000
H = 128
G = 16

NC = 2
NS = 16
NW = NC * NS
EP = E // NW
CH = 80
NJ = EP // CH
NP = 10240
EH = E // 2
CHh = 40
NJh = (EH // NW) // CHh
RS = NP // NS

TE = 1280
GE = E // TE
TN = 1000
GN = N // TN
TBC = 8000

_f32 = jnp.float32


def _mesh():
    return plsc.VectorSubcoreMesh(core_axis_name="c", subcore_axis_name="s",
                                  num_cores=NC, num_subcores=NS)


@functools.lru_cache(maxsize=None)
def _get_sc_gather(es, ch, nj):
    eps = es // NW

    @functools.partial(
        pl.kernel,
        out_type=(jax.ShapeDtypeStruct((es, H), _f32),
                  jax.ShapeDtypeStruct((es, H), _f32)),
        mesh=_mesh(),
        scratch_types=[
            pltpu.VMEM((nj, ch), jnp.int32),
            pltpu.VMEM((nj, ch), jnp.int32),
            pltpu.VMEM((ch, H), _f32),
            pltpu.VMEM((ch, H), _f32),
            pltpu.SemaphoreType.DMA,
            pltpu.SemaphoreType.DMA,
        ],
    )
    def k(xi_hbm, xj_hbm, dst3_hbm, src3_hbm, outa_hbm, outb_hbm,
          idxd, idxs, bufa, bufb, sema, semb):
        c = lax.axis_index("c")
        s = lax.axis_index("s")
        wid = s * NC + c
        base = wid * eps
        pltpu.sync_copy(dst3_hbm.at[wid], idxd)
        pltpu.sync_copy(src3_hbm.at[wid], idxs)

        def body(j, carry):
            a = pltpu.async_copy(xi_hbm.at[idxd.at[j]], bufa, sema)
            b = pltpu.async_copy(xj_hbm.at[idxs.at[j]], bufb, semb)
            a.wait()
            b.wait()
            pltpu.sync_copy(bufa, outa_hbm.at[pl.ds(base + j * ch, ch)])
            pltpu.sync_copy(bufb, outb_hbm.at[pl.ds(base + j * ch, ch)])
            return carry

        lax.fori_loop(0, nj, body, 0)

    return k


def _sc_gather2(xi, xj, dst3, src3):
    es = dst3.shape[0] * dst3.shape[1] * dst3.shape[2]
    return _get_sc_gather(es, dst3.shape[2], dst3.shape[1])(
        xi, xj, dst3, src3)


@functools.lru_cache(maxsize=None)
def _get_sc_scatter(es, ch, nj):
    eps = es // NW

    @functools.partial(
        pl.kernel,
        out_type=jax.ShapeDtypeStruct((NC, NP, H), _f32),
        mesh=_mesh(),
        scratch_types=[
            pltpu.VMEM((nj, ch), jnp.int32),
            pltpu.VMEM((ch, H), _f32),
            pltpu.VMEM_SHARED((NP, H), _f32),
        ],
    )
    def k(vals_hbm, idx3_hbm, zeros_hbm, out_hbm, idxv, chunk, acc):
        c = lax.axis_index("c")
        s = lax.axis_index("s")
        wid = s * NC + c
        base = wid * eps
        pltpu.sync_copy(idx3_hbm.at[wid], idxv)
        pltpu.sync_copy(zeros_hbm.at[pl.ds(s * RS, RS)],
                        acc.at[pl.ds(s * RS, RS)])
        plsc.subcore_barrier()

        def body(j, carry):
            pltpu.sync_copy(vals_hbm.at[pl.ds(base + j * ch, ch)], chunk)
            pltpu.sync_copy(chunk, acc.at[idxv.at[j]], add=True)
            return carry

        lax.fori_loop(0, nj, body, 0)
        plsc.subcore_barrier()
        pltpu.sync_copy(acc.at[pl.ds(s * RS, RS)],
                        out_hbm.at[c, pl.ds(s * RS, RS)])

    return k


def _sc_scatter(vals, idx3, zeros):
    es = idx3.shape[0] * idx3.shape[1] * idx3.shape[2]
    return _get_sc_scatter(es, idx3.shape[2], idx3.shape[1])(
        vals, idx3, zeros)


@functools.lru_cache(maxsize=None)
def _get_sc_hist():
    @functools.partial(
        pl.kernel,
        out_type=jax.ShapeDtypeStruct((NW, 8, H), _f32),
        mesh=_mesh(),
        scratch_types=[
            pltpu.VMEM((NJ, CH), jnp.int32),
            pltpu.VMEM((CH, H), _f32),
            pltpu.VMEM((8, H), _f32),
            pltpu.SemaphoreType.DMA,
        ],
    )
    def k(tab_hbm, src3_hbm, zeros8_hbm, out_hbm, idxs, buf, accv, sem):
        c = lax.axis_index("c")
        s = lax.axis_index("s")
        wid = s * NC + c
        pltpu.sync_copy(src3_hbm.at[wid], idxs)
        pltpu.sync_copy(zeros8_hbm, accv)

        def body(j, acc):
            pltpu.async_copy(tab_hbm.at[idxs.at[j]], buf, sem).wait()
            for r in range(CH):
                acc = acc + buf[r, pl.ds(0, 16)]
            return acc

        acc = lax.fori_loop(0, NJ, body, jnp.zeros((16,), _f32))
        accv[0, pl.ds(0, 16)] = acc
        pltpu.sync_copy(accv, out_hbm.at[wid])

    return k


def _sc_hist(onehot128, src3, zeros8):
    return _get_sc_hist()(onehot128, src3, zeros8)


def _dot(a, b):
    return jnp.dot(a, b, preferred_element_type=_f32)


def _dotx(a, b):
    return jnp.dot(a, b, preferred_element_type=_f32,
                   precision=lax.Precision.HIGHEST)


def _wspec(shape):
    return pl.BlockSpec(shape, lambda i: (0,) * len(shape))


def _tc_node0(xpad, w1, w2, w3, bmat, wi, wj):
    def body(x_ref, w1_ref, w2_ref, w3_ref, b_ref, wi_ref, wj_ref,
             x0_ref, xi_ref, xj_ref):
        h = jnp.maximum(x_ref[:, 0:1] * w1_ref[0:1, :] + b_ref[0], 0.0)
        h = jnp.maximum(_dot(h, w2_ref[...]) + b_ref[1], 0.0)
        x0 = _dot(h, w3_ref[...]) + b_ref[2]
        x0_ref[...] = x0
        xi_ref[...] = _dot(x0, wi_ref[...])
        xj_ref[...] = _dot(x0, wj_ref[...])

    return pl.pallas_call(
        body,
        grid=(GN,),
        in_specs=[
            pl.BlockSpec((TN, 8), lambda i: (i, 0)),
            _wspec((8, H)), _wspec((H, H)), _wspec((H, H)),
            _wspec((8, H)), _wspec((H, H)), _wspec((H, H)),
        ],
        out_specs=[pl.BlockSpec((TN, H), lambda i: (i, 0))] * 3,
        out_shape=[jax.ShapeDtypeStruct((N, H), _f32)] * 3,
    )(xpad, w1, w2, w3, bmat, wi, wj)


def _tc_edge1(attr8, ga, gb, wa1, wa2, wa3, we1ef, we2, we3, gfe, starts8,
              ends8, bmat, pos_off):
    def body(a_ref, ga_ref, gb_ref, wa1_ref, wa2_ref, wa3_ref, wef_ref,
             we2_ref, we3_ref, gfe_ref, st_ref, en_ref, b_ref,
             ef0_ref, msg_ref):
        i = pl.program_id(0)
        a = a_ref[...]
        h = jnp.maximum(_dot(a, wa1_ref[...]) + b_ref[0], 0.0)
        h = jnp.maximum(_dot(h, wa2_ref[...]) + b_ref[1], 0.0)
        ef0 = _dot(h, wa3_ref[...]) + b_ref[2]
        ef0_ref[...] = ef0
        pos = (lax.broadcasted_iota(jnp.int32, (TE, 1), 0)
               + (i * TE + pos_off)).astype(_f32)
        st = st_ref[0:1, 0:G]
        en = en_ref[0:1, 0:G]
        oh = ((pos >= st) & (pos < en)).astype(_f32)
        gfm = _dotx(oh, gfe_ref[...])
        h1 = jnp.maximum(_dot(ef0, wef_ref[...]) + (ga_ref[...] + gb_ref[...])
                         + gfm + b_ref[3], 0.0)
        h2 = jnp.maximum(_dot(h1, we2_ref[...]) + b_ref[4], 0.0)
        msg_ref[...] = _dot(h2, we3_ref[...]) + b_ref[5]

    es = attr8.shape[0]
    return pl.pallas_call(
        body,
        grid=(es // TE,),
        in_specs=[
            pl.BlockSpec((TE, 8), lambda i: (i, 0)),
            pl.BlockSpec((TE, H), lambda i: (i, 0)),
            pl.BlockSpec((TE, H), lambda i: (i, 0)),
            _wspec((8, H)), _wspec((H, H)), _wspec((H, H)),
            _wspec((H, H)), _wspec((H, H)), _wspec((H, H)),
            _wspec((G, H)), _wspec((8, 32)), _wspec((8, 32)),
            _wspec((8, H)),
        ],
        out_specs=[pl.BlockSpec((TE, H), lambda i: (i, 0))] * 2,
        out_shape=[jax.ShapeDtypeStruct((es, H), _f32)] * 2,
    )(attr8, ga, gb, wa1, wa2, wa3, we1ef, we2, we3, gfe, starts8, ends8,
      bmat)


def _tc_node1(x0, sega, segb, onehotn, wx, wa, gfn, w2, w3, bmat, wi, wj):
    def body(x_ref, sega_ref, segb_ref, oh_ref, wx_ref, wa_ref, gfn_ref,
             w2_ref, w3_ref, b_ref, wi_ref, wj_ref, x1_ref, xi_ref, xj_ref):
        x = x_ref[...]
        aggr = (sega_ref[0] + sega_ref[1]) + (segb_ref[0] + segb_ref[1])
        gfm = _dotx(oh_ref[...], gfn_ref[...])
        h = jnp.maximum(_dot(x, wx_ref[...]) + _dot(aggr, wa_ref[...])
                        + gfm + b_ref[0], 0.0)
        h = jnp.maximum(_dot(h, w2_ref[...]) + b_ref[1], 0.0)
        x1 = x + _dot(h, w3_ref[...]) + b_ref[2]
        x1_ref[...] = x1
        xi_ref[...] = _dot(x1, wi_ref[...])
        xj_ref[...] = _dot(x1, wj_ref[...])

    return pl.pallas_call(
        body,
        grid=(GN,),
        in_specs=[
            pl.BlockSpec((TN, H), lambda i: (i, 0)),
            pl.BlockSpec((NC, TN, H), lambda i: (0, i, 0)),
            pl.BlockSpec((NC, TN, H), lambda i: (0, i, 0)),
            pl.BlockSpec((TN, G), lambda i: (i, 0)),
            _wspec((H, H)), _wspec((H, H)), _wspec((G, H)),
            _wspec((H, H)), _wspec((H, H)), _wspec((8, H)),
            _wspec((H, H)), _wspec((H, H)),
        ],
        out_specs=[pl.BlockSpec((TN, H), lambda i: (i, 0))] * 3,
        out_shape=[jax.ShapeDtypeStruct((N, H), _f32)] * 3,
    )(x0, sega, segb, onehotn, wx, wa, gfn, w2, w3, bmat, wi, wj)


def _tc_edge2(ef0, msg1, ga, gb, we1ef, we2, we3, gfe, starts8, ends8, bmat,
              pos_off):
    def body(ef0_ref, m1_ref, ga_ref, gb_ref, wef_ref, we2_ref, we3_ref,
             gfe_ref, st_ref, en_ref, b_ref, msg_ref):
        i = pl.program_id(0)
        ef1 = ef0_ref[...] + m1_ref[...]
        pos = (lax.broadcasted_iota(jnp.int32, (TE, 1), 0)
               + (i * TE + pos_off)).astype(_f32)
        st = st_ref[0:1, 0:G]
        en = en_ref[0:1, 0:G]
        oh = ((pos >= st) & (pos < en)).astype(_f32)
        gfm = _dotx(oh, gfe_ref[...])
        h1 = jnp.maximum(_dot(ef1, wef_ref[...]) + (ga_ref[...] + gb_ref[...])
                         + gfm + b_ref[0], 0.0)
        h2 = jnp.maximum(_dot(h1, we2_ref[...]) + b_ref[1], 0.0)
        msg_ref[...] = _dot(h2, we3_ref[...]) + b_ref[2]

    es = ef0.shape[0]
    return pl.pallas_call(
        body,
        grid=(es // TE,),
        in_specs=[
            pl.BlockSpec((TE, H), lambda i: (i, 0)),
            pl.BlockSpec((TE, H), lambda i: (i, 0)),
            pl.BlockSpec((TE, H), lambda i: (i, 0)),
            pl.BlockSpec((TE, H), lambda i: (i, 0)),
            _wspec((H, H)), _wspec((H, H)), _wspec((H, H)),
            _wspec((G, H)), _wspec((8, 32)), _wspec((8, 32)),
            _wspec((8, H)),
        ],
        out_specs=pl.BlockSpec((TE, H), lambda i: (i, 0)),
        out_shape=jax.ShapeDtypeStruct((es, H), _f32),
    )(ef0, msg1, ga, gb, we1ef, we2, we3, gfe, starts8, ends8, bmat)


def _tc_node2(x1, sega, segb, onehotn, wx, wa, gfn, w2, w3, bmat, wact1,
              wact2, wact3p, bact):
    def body(x_ref, sega_ref, segb_ref, oh_ref, wx_ref, wa_ref, gfn_ref,
             w2_ref, w3_ref, b_ref, a1_ref, a2_ref, a3_ref, ba_ref, out_ref):
        x = x_ref[...]
        aggr = (sega_ref[0] + sega_ref[1]) + (segb_ref[0] + segb_ref[1])
        gfm = _dotx(oh_ref[...], gfn_ref[...])
        h = jnp.maximum(_dot(x, wx_ref[...]) + _dot(aggr, wa_ref[...])
                        + gfm + b_ref[0], 0.0)
        h = jnp.maximum(_dot(h, w2_ref[...]) + b_ref[1], 0.0)
        x2 = x + _dot(h, w3_ref[...]) + b_ref[2]
        a = jnp.maximum(_dot(x2, a1_ref[...]) + ba_ref[0], 0.0)
        a = jnp.maximum(_dot(a, a2_ref[...]) + ba_ref[1], 0.0)
        out_ref[...] = _dot(a, a3_ref[...]) + ba_ref[2]

    return pl.pallas_call(
        body,
        grid=(GN,),
        in_specs=[
            pl.BlockSpec((TN, H), lambda i: (i, 0)),
            pl.BlockSpec((NC, TN, H), lambda i: (0, i, 0)),
            pl.BlockSpec((NC, TN, H), lambda i: (0, i, 0)),
            pl.BlockSpec((TN, G), lambda i: (i, 0)),
            _wspec((H, H)), _wspec((H, H)), _wspec((G, H)),
            _wspec((H, H)), _wspec((H, H)), _wspec((8, H)),
            _wspec((H, H)), _wspec((H, H)), _wspec((H, H)),
            _wspec((8, H)),
        ],
        out_specs=pl.BlockSpec((TN, H), lambda i: (i, 0)),
        out_shape=jax.ShapeDtypeStruct((N, H), _f32),
    )(x1, sega, segb, onehotn, wx, wa, gfn, w2, w3, bmat, wact1, wact2,
      wact3p, bact)


def _bias_mat(*bs):
    m = jnp.zeros((8, H), _f32)
    for r, b in enumerate(bs):
        m = m.at[r, : b.shape[0]].set(b)
    return m


def _pad_rows(w, rows):
    return jnp.zeros((rows, w.shape[1]), _f32).at[: w.shape[0]].set(w)


def kernel(contact_node, contact_force, initial_edge_delta, length,
           parent2child, branch, stiffness, edge_index, batch, params):
    src = edge_index[0].astype(jnp.int32)
    dst = edge_index[1].astype(jnp.int32)
    src3 = src.reshape(NW, NJ, CH)
    dst3a = dst[:EH].reshape(NW, NJh, CHh)
    dst3b = dst[EH:].reshape(NW, NJh, CHh)
    src3a = src[:EH].reshape(NW, NJh, CHh)
    src3b = src[EH:].reshape(NW, NJh, CHh)

    attr8 = jnp.concatenate(
        [initial_edge_delta, length[:, None], parent2child[:, None],
         branch[:, None], stiffness[:, None], jnp.zeros((E, 1), _f32)],
        axis=1)
    xpad = jnp.pad(contact_node[:, None].astype(_f32), ((0, 0), (0, 7)))
    onehotn = (batch[:, None] == jnp.arange(G, dtype=batch.dtype)[None, :]
               ).astype(_f32)
    zerosH = jnp.zeros((NP, H), _f32)
    gf = contact_force.reshape(G, 3)

    (wn01, bn01), (wn02, bn02), (wn03, bn03) = params['node_in']
    (wa1, ba1), (wa2, ba2), (wa3, ba3) = params['edge_in']
    (wc1, bc1), (wc2, bc2), (wc3, bc3) = params['node_action']
    in1, in2 = params['IN'][0], params['IN'][1]
    (e11, be11), (e12, be12), (e13, be13) = in1['lin_edge']
    (n11, bn11), (n12, bn12), (n13, bn13) = in1['lin_node']
    (e21, be21), (e22, be22), (e23, be23) = in2['lin_edge']
    (n21, bn21), (n22, bn22), (n23, bn23) = in2['lin_node']

    w1i_1, w1j_1, w1e_1, w1g_1 = e11[:H], e11[H:2*H], e11[2*H:3*H], e11[3*H:]
    w1i_2, w1j_2, w1e_2, w1g_2 = e21[:H], e21[H:2*H], e21[2*H:3*H], e21[3*H:]
    nx1, na1, ng1 = n11[:H], n11[H:2*H], n11[2*H:]
    nx2, na2, ng2 = n21[:H], n21[H:2*H], n21[2*H:]

    gfe1 = _dot(gf, w1g_1)
    gfe2 = _dot(gf, w1g_2)
    gfn1 = _dot(gf, ng1)
    gfn2 = _dot(gf, ng2)

    onehot128 = jnp.pad(onehotn, ((0, 0), (0, H - G)))
    hist = _sc_hist(onehot128, src3, jnp.zeros((8, H), _f32))
    counts = jnp.sum(hist, axis=(0, 1))[:G]
    ends_v = jnp.cumsum(counts)
    starts_v = ends_v - counts
    starts8 = jnp.broadcast_to(
        jnp.zeros((32,), _f32).at[:G].set(starts_v), (8, 32))
    ends8 = jnp.broadcast_to(
        jnp.zeros((32,), _f32).at[:G].set(ends_v), (8, 32))

    x0, xi1, xj1 = _tc_node0(
        xpad, _pad_rows(wn01, 8), wn02, wn03,
        _bias_mat(bn01, bn02, bn03), w1i_1, w1j_1)

    eb1 = _bias_mat(ba1, ba2, ba3, be11, be12, be13)
    ga1a, gb1a = _sc_gather2(xi1, xj1, dst3a, src3a)
    ga1b, gb1b = _sc_gather2(xi1, xj1, dst3b, src3b)
    ef0a, msg1a = _tc_edge1(
        attr8[:EH], ga1a, gb1a, _pad_rows(wa1, 8), wa2, wa3, w1e_1, e12,
        e13, gfe1, starts8, ends8, eb1, 0)
    ef0b, msg1b = _tc_edge1(
        attr8[EH:], ga1b, gb1b, _pad_rows(wa1, 8), wa2, wa3, w1e_1, e12,
        e13, gfe1, starts8, ends8, eb1, EH)
    seg1a = _sc_scatter(msg1a, dst3a, zerosH)
    seg1b = _sc_scatter(msg1b, dst3b, zerosH)
    x1, xi2, xj2 = _tc_node1(
        x0, seg1a, seg1b, onehotn, nx1, na1, gfn1, n12, n13,
        _bias_mat(bn11, bn12, bn13), w1i_2, w1j_2)

    eb2 = _bias_mat(be21, be22, be23)
    ga2a, gb2a = _sc_gather2(xi2, xj2, dst3a, src3a)
    ga2b, gb2b = _sc_gather2(xi2, xj2, dst3b, src3b)
    msg2a = _tc_edge2(ef0a, msg1a, ga2a, gb2a, w1e_2, e22, e23, gfe2,
                      starts8, ends8, eb2, 0)
    msg2b = _tc_edge2(ef0b, msg1b, ga2b, gb2b, w1e_2, e22, e23, gfe2,
                      starts8, ends8, eb2, EH)
    seg2a = _sc_scatter(msg2a, dst3a, zerosH)
    seg2b = _sc_scatter(msg2b, dst3b, zerosH)

    wc3p = jnp.zeros((H, H), _f32).at[:, :3].set(wc3)
    bc3p = jnp.zeros((H,), _f32).at[:3].set(bc3)
    out = _tc_node2(
        x1, seg2a, seg2b, onehotn, nx2, na2, gfn2, n22, n23,
        _bias_mat(bn21, bn22, bn23), wc1, wc2, wc3p,
        _bias_mat(bc1, bc2, bc3p))
    return out[:, :3]


# speedup vs baseline: 1.2679x; 1.0012x over previous
import functools

import jax
import jax.numpy as jnp
from jax import lax
from jax.experimental import pallas as pl
from jax.experimental.pallas import tpu as pltpu
from jax.experimental.pallas import tpu_sc as plsc

N = 10000
E = 320000
H = 128
G = 16

NC = 2
NS = 16
NW = NC * NS
EP = E // NW
CH = 80
NJ = EP // CH
NP = 10240
EH = E // 2
CHh = 40
NJh = (EH // NW) // CHh
RS = NP // NS

TE = 1280
GE = E // TE
TN = 1000
GN = N // TN
TBC = 8000

_f32 = jnp.float32


def _mesh():
    return plsc.VectorSubcoreMesh(core_axis_name="c", subcore_axis_name="s",
                                  num_cores=NC, num_subcores=NS)


@functools.lru_cache(maxsize=None)
def _get_sc_gather(es, ch, nj):
    eps = es // NW

    @functools.partial(
        pl.kernel,
        out_type=(jax.ShapeDtypeStruct((es, H), _f32),
                  jax.ShapeDtypeStruct((es, H), _f32)),
        mesh=_mesh(),
        scratch_types=[
            pltpu.VMEM((nj, ch), jnp.int32),
            pltpu.VMEM((nj, ch), jnp.int32),
            pltpu.VMEM((ch, H), _f32),
            pltpu.VMEM((ch, H), _f32),
            pltpu.SemaphoreType.DMA,
            pltpu.SemaphoreType.DMA,
        ],
    )
    def k(xi_hbm, xj_hbm, dst3_hbm, src3_hbm, outa_hbm, outb_hbm,
          idxd, idxs, bufa, bufb, sema, semb):
        c = lax.axis_index("c")
        s = lax.axis_index("s")
        wid = s * NC + c
        base = wid * eps
        pltpu.sync_copy(dst3_hbm.at[wid], idxd)
        pltpu.sync_copy(src3_hbm.at[wid], idxs)

        def body(j, carry):
            a = pltpu.async_copy(xi_hbm.at[idxd.at[j]], bufa, sema)
            b = pltpu.async_copy(xj_hbm.at[idxs.at[j]], bufb, semb)
            a.wait()
            b.wait()
            pltpu.sync_copy(bufa, outa_hbm.at[pl.ds(base + j * ch, ch)])
            pltpu.sync_copy(bufb, outb_hbm.at[pl.ds(base + j * ch, ch)])
            return carry

        lax.fori_loop(0, nj, body, 0)

    return k


def _sc_gather2(xi, xj, dst3, src3):
    es = dst3.shape[0] * dst3.shape[1] * dst3.shape[2]
    return _get_sc_gather(es, dst3.shape[2], dst3.shape[1])(
        xi, xj, dst3, src3)


@functools.lru_cache(maxsize=None)
def _get_sc_scatter(es, ch, nj):
    eps = es // NW

    @functools.partial(
        pl.kernel,
        out_type=jax.ShapeDtypeStruct((NC, NP, H), _f32),
        mesh=_mesh(),
        scratch_types=[
            pltpu.VMEM((nj, ch), jnp.int32),
            pltpu.VMEM((ch, H), _f32),
            pltpu.VMEM_SHARED((NP, H), _f32),
        ],
    )
    def k(vals_hbm, idx3_hbm, zeros_hbm, out_hbm, idxv, chunk, acc):
        c = lax.axis_index("c")
        s = lax.axis_index("s")
        wid = s * NC + c
        base = wid * eps
        pltpu.sync_copy(idx3_hbm.at[wid], idxv)
        pltpu.sync_copy(zeros_hbm.at[pl.ds(s * RS, RS)],
                        acc.at[pl.ds(s * RS, RS)])
        plsc.subcore_barrier()

        def body(j, carry):
            pltpu.sync_copy(vals_hbm.at[pl.ds(base + j * ch, ch)], chunk)
            pltpu.sync_copy(chunk, acc.at[idxv.at[j]], add=True)
            return carry

        lax.fori_loop(0, nj, body, 0)
        plsc.subcore_barrier()
        pltpu.sync_copy(acc.at[pl.ds(s * RS, RS)],
                        out_hbm.at[c, pl.ds(s * RS, RS)])

    return k


def _sc_scatter(vals, idx3, zeros):
    es = idx3.shape[0] * idx3.shape[1] * idx3.shape[2]
    return _get_sc_scatter(es, idx3.shape[2], idx3.shape[1])(
        vals, idx3, zeros)


@functools.lru_cache(maxsize=None)
def _get_sc_hist():
    @functools.partial(
        pl.kernel,
        out_type=jax.ShapeDtypeStruct((NW, 8, H), _f32),
        mesh=_mesh(),
        scratch_types=[
            pltpu.VMEM((NJ, CH), jnp.int32),
            pltpu.VMEM((CH, H), _f32),
            pltpu.VMEM((8, H), _f32),
            pltpu.SemaphoreType.DMA,
        ],
    )
    def k(tab_hbm, src3_hbm, zeros8_hbm, out_hbm, idxs, buf, accv, sem):
        c = lax.axis_index("c")
        s = lax.axis_index("s")
        wid = s * NC + c
        pltpu.sync_copy(src3_hbm.at[wid], idxs)
        pltpu.sync_copy(zeros8_hbm, accv)

        def body(j, acc):
            pltpu.async_copy(tab_hbm.at[idxs.at[j]], buf, sem).wait()
            for r in range(CH):
                acc = acc + buf[r, pl.ds(0, 16)]
            return acc

        acc = lax.fori_loop(0, NJ, body, jnp.zeros((16,), _f32))
        accv[0, pl.ds(0, 16)] = acc
        pltpu.sync_copy(accv, out_hbm.at[wid])

    return k


def _sc_hist(onehot128, src3, zeros8):
    return _get_sc_hist()(onehot128, src3, zeros8)


def _dot(a, b):
    return jnp.dot(a, b, preferred_element_type=_f32)


def _dotx(a, b):
    return jnp.dot(a, b, preferred_element_type=_f32,
                   precision=lax.Precision.HIGHEST)


def _wspec(shape):
    return pl.BlockSpec(shape, lambda i: (0,) * len(shape))


def _tc_node0(xpad, w1, w2, w3, bmat, wi, wj):
    def body(x_ref, w1_ref, w2_ref, w3_ref, b_ref, wi_ref, wj_ref,
             x0_ref, xi_ref, xj_ref):
        h = jnp.maximum(x_ref[:, 0:1] * w1_ref[0:1, :] + b_ref[0], 0.0)
        h = jnp.maximum(_dot(h, w2_ref[...]) + b_ref[1], 0.0)
        x0 = _dot(h, w3_ref[...]) + b_ref[2]
        x0_ref[...] = x0
        xi_ref[...] = _dot(x0, wi_ref[...])
        xj_ref[...] = _dot(x0, wj_ref[...])

    return pl.pallas_call(
        body,
        grid=(GN,),
        in_specs=[
            pl.BlockSpec((TN, 8), lambda i: (i, 0)),
            _wspec((8, H)), _wspec((H, H)), _wspec((H, H)),
            _wspec((8, H)), _wspec((H, H)), _wspec((H, H)),
        ],
        out_specs=[pl.BlockSpec((TN, H), lambda i: (i, 0))] * 3,
        out_shape=[jax.ShapeDtypeStruct((N, H), _f32)] * 3,
    )(xpad, w1, w2, w3, bmat, wi, wj)


def _tc_edge1(attr8, ga, gb, wa1, wa2, wa3, we1ef, we2, we3, gfe, starts8,
              ends8, bmat, pos_off):
    def body(a_ref, ga_ref, gb_ref, wa1_ref, wa2_ref, wa3_ref, wef_ref,
             we2_ref, we3_ref, gfe_ref, st_ref, en_ref, b_ref,
             ef0_ref, msg_ref):
        i = pl.program_id(0)
        a = a_ref[...]
        h = jnp.maximum(_dot(a, wa1_ref[...]) + b_ref[0], 0.0)
        h = jnp.maximum(_dot(h, wa2_ref[...]) + b_ref[1], 0.0)
        ef0 = _dot(h, wa3_ref[...]) + b_ref[2]
        ef0_ref[...] = ef0
        pos = (lax.broadcasted_iota(jnp.int32, (TE, 1), 0)
               + (i * TE + pos_off)).astype(_f32)
        st = st_ref[0:1, 0:G]
        en = en_ref[0:1, 0:G]
        oh = ((pos >= st) & (pos < en)).astype(_f32)
        gfm = _dotx(oh, gfe_ref[...])
        h1 = jnp.maximum((ga_ref[...] + gb_ref[...]) + _dot(ef0, wef_ref[...])
                         + gfm + b_ref[3], 0.0)
        h2 = jnp.maximum(_dot(h1, we2_ref[...]) + b_ref[4], 0.0)
        msg_ref[...] = _dot(h2, we3_ref[...]) + b_ref[5]

    es = attr8.shape[0]
    return pl.pallas_call(
        body,
        grid=(es // TE,),
        in_specs=[
            pl.BlockSpec((TE, 8), lambda i: (i, 0)),
            pl.BlockSpec((TE, H), lambda i: (i, 0)),
            pl.BlockSpec((TE, H), lambda i: (i, 0)),
            _wspec((8, H)), _wspec((H, H)), _wspec((H, H)),
            _wspec((H, H)), _wspec((H, H)), _wspec((H, H)),
            _wspec((G, H)), _wspec((8, 32)), _wspec((8, 32)),
            _wspec((8, H)),
        ],
        out_specs=[pl.BlockSpec((TE, H), lambda i: (i, 0))] * 2,
        out_shape=[jax.ShapeDtypeStruct((es, H), _f32)] * 2,
    )(attr8, ga, gb, wa1, wa2, wa3, we1ef, we2, we3, gfe, starts8, ends8,
      bmat)


def _tc_node1(x0, sega, segb, onehotn, wx, wa, gfn, w2, w3, bmat, wi, wj):
    def body(x_ref, sega_ref, segb_ref, oh_ref, wx_ref, wa_ref, gfn_ref,
             w2_ref, w3_ref, b_ref, wi_ref, wj_ref, x1_ref, xi_ref, xj_ref):
        x = x_ref[...]
        aggr = (sega_ref[0] + sega_ref[1]) + (segb_ref[0] + segb_ref[1])
        gfm = _dotx(oh_ref[...], gfn_ref[...])
        h = jnp.maximum(_dot(x, wx_ref[...]) + _dot(aggr, wa_ref[...])
                        + gfm + b_ref[0], 0.0)
        h = jnp.maximum(_dot(h, w2_ref[...]) + b_ref[1], 0.0)
        x1 = x + _dot(h, w3_ref[...]) + b_ref[2]
        x1_ref[...] = x1
        xi_ref[...] = _dot(x1, wi_ref[...])
        xj_ref[...] = _dot(x1, wj_ref[...])

    return pl.pallas_call(
        body,
        grid=(GN,),
        in_specs=[
            pl.BlockSpec((TN, H), lambda i: (i, 0)),
            pl.BlockSpec((NC, TN, H), lambda i: (0, i, 0)),
            pl.BlockSpec((NC, TN, H), lambda i: (0, i, 0)),
            pl.BlockSpec((TN, G), lambda i: (i, 0)),
            _wspec((H, H)), _wspec((H, H)), _wspec((G, H)),
            _wspec((H, H)), _wspec((H, H)), _wspec((8, H)),
            _wspec((H, H)), _wspec((H, H)),
        ],
        out_specs=[pl.BlockSpec((TN, H), lambda i: (i, 0))] * 3,
        out_shape=[jax.ShapeDtypeStruct((N, H), _f32)] * 3,
    )(x0, sega, segb, onehotn, wx, wa, gfn, w2, w3, bmat, wi, wj)


def _tc_edge2(ef0, msg1, ga, gb, we1ef, we2, we3, gfe, starts8, ends8, bmat,
              pos_off):
    def body(ef0_ref, m1_ref, ga_ref, gb_ref, wef_ref, we2_ref, we3_ref,
             gfe_ref, st_ref, en_ref, b_ref, msg_ref):
        i = pl.program_id(0)
        ef1 = ef0_ref[...] + m1_ref[...]
        pos = (lax.broadcasted_iota(jnp.int32, (TE, 1), 0)
               + (i * TE + pos_off)).astype(_f32)
        st = st_ref[0:1, 0:G]
        en = en_ref[0:1, 0:G]
        oh = ((pos >= st) & (pos < en)).astype(_f32)
        gfm = _dotx(oh, gfe_ref[...])
        h1 = jnp.maximum((ga_ref[...] + gb_ref[...]) + _dot(ef1, wef_ref[...])
                         + gfm + b_ref[0], 0.0)
        h2 = jnp.maximum(_dot(h1, we2_ref[...]) + b_ref[1], 0.0)
        msg_ref[...] = _dot(h2, we3_ref[...]) + b_ref[2]

    es = ef0.shape[0]
    return pl.pallas_call(
        body,
        grid=(es // TE,),
        in_specs=[
            pl.BlockSpec((TE, H), lambda i: (i, 0)),
            pl.BlockSpec((TE, H), lambda i: (i, 0)),
            pl.BlockSpec((TE, H), lambda i: (i, 0)),
            pl.BlockSpec((TE, H), lambda i: (i, 0)),
            _wspec((H, H)), _wspec((H, H)), _wspec((H, H)),
            _wspec((G, H)), _wspec((8, 32)), _wspec((8, 32)),
            _wspec((8, H)),
        ],
        out_specs=pl.BlockSpec((TE, H), lambda i: (i, 0)),
        out_shape=jax.ShapeDtypeStruct((es, H), _f32),
    )(ef0, msg1, ga, gb, we1ef, we2, we3, gfe, starts8, ends8, bmat)


def _tc_node2(x1, sega, segb, onehotn, wx, wa, gfn, w2, w3, bmat, wact1,
              wact2, wact3p, bact):
    def body(x_ref, sega_ref, segb_ref, oh_ref, wx_ref, wa_ref, gfn_ref,
             w2_ref, w3_ref, b_ref, a1_ref, a2_ref, a3_ref, ba_ref, out_ref):
        x = x_ref[...]
        aggr = (sega_ref[0] + sega_ref[1]) + (segb_ref[0] + segb_ref[1])
        gfm = _dotx(oh_ref[...], gfn_ref[...])
        h = jnp.maximum(_dot(x, wx_ref[...]) + _dot(aggr, wa_ref[...])
                        + gfm + b_ref[0], 0.0)
        h = jnp.maximum(_dot(h, w2_ref[...]) + b_ref[1], 0.0)
        x2 = x + _dot(h, w3_ref[...]) + b_ref[2]
        a = jnp.maximum(_dot(x2, a1_ref[...]) + ba_ref[0], 0.0)
        a = jnp.maximum(_dot(a, a2_ref[...]) + ba_ref[1], 0.0)
        out_ref[...] = _dot(a, a3_ref[...]) + ba_ref[2]

    return pl.pallas_call(
        body,
        grid=(GN,),
        in_specs=[
            pl.BlockSpec((TN, H), lambda i: (i, 0)),
            pl.BlockSpec((NC, TN, H), lambda i: (0, i, 0)),
            pl.BlockSpec((NC, TN, H), lambda i: (0, i, 0)),
            pl.BlockSpec((TN, G), lambda i: (i, 0)),
            _wspec((H, H)), _wspec((H, H)), _wspec((G, H)),
            _wspec((H, H)), _wspec((H, H)), _wspec((8, H)),
            _wspec((H, H)), _wspec((H, H)), _wspec((H, H)),
            _wspec((8, H)),
        ],
        out_specs=pl.BlockSpec((TN, H), lambda i: (i, 0)),
        out_shape=jax.ShapeDtypeStruct((N, H), _f32),
    )(x1, sega, segb, onehotn, wx, wa, gfn, w2, w3, bmat, wact1, wact2,
      wact3p, bact)


def _bias_mat(*bs):
    m = jnp.zeros((8, H), _f32)
    for r, b in enumerate(bs):
        m = m.at[r, : b.shape[0]].set(b)
    return m


def _pad_rows(w, rows):
    return jnp.zeros((rows, w.shape[1]), _f32).at[: w.shape[0]].set(w)


def kernel(contact_node, contact_force, initial_edge_delta, length,
           parent2child, branch, stiffness, edge_index, batch, params):
    src = edge_index[0].astype(jnp.int32)
    dst = edge_index[1].astype(jnp.int32)
    src3 = src.reshape(NW, NJ, CH)
    dst3a = dst[:EH].reshape(NW, NJh, CHh)
    dst3b = dst[EH:].reshape(NW, NJh, CHh)
    src3a = src[:EH].reshape(NW, NJh, CHh)
    src3b = src[EH:].reshape(NW, NJh, CHh)

    attr8 = jnp.concatenate(
        [initial_edge_delta, length[:, None], parent2child[:, None],
         branch[:, None], stiffness[:, None], jnp.zeros((E, 1), _f32)],
        axis=1)
    xpad = jnp.pad(contact_node[:, None].astype(_f32), ((0, 0), (0, 7)))
    onehotn = (batch[:, None] == jnp.arange(G, dtype=batch.dtype)[None, :]
               ).astype(_f32)
    zerosH = jnp.zeros((NP, H), _f32)
    gf = contact_force.reshape(G, 3)

    (wn01, bn01), (wn02, bn02), (wn03, bn03) = params['node_in']
    (wa1, ba1), (wa2, ba2), (wa3, ba3) = params['edge_in']
    (wc1, bc1), (wc2, bc2), (wc3, bc3) = params['node_action']
    in1, in2 = params['IN'][0], params['IN'][1]
    (e11, be11), (e12, be12), (e13, be13) = in1['lin_edge']
    (n11, bn11), (n12, bn12), (n13, bn13) = in1['lin_node']
    (e21, be21), (e22, be22), (e23, be23) = in2['lin_edge']
    (n21, bn21), (n22, bn22), (n23, bn23) = in2['lin_node']

    w1i_1, w1j_1, w1e_1, w1g_1 = e11[:H], e11[H:2*H], e11[2*H:3*H], e11[3*H:]
    w1i_2, w1j_2, w1e_2, w1g_2 = e21[:H], e21[H:2*H], e21[2*H:3*H], e21[3*H:]
    nx1, na1, ng1 = n11[:H], n11[H:2*H], n11[2*H:]
    nx2, na2, ng2 = n21[:H], n21[H:2*H], n21[2*H:]

    gfe1 = _dot(gf, w1g_1)
    gfe2 = _dot(gf, w1g_2)
    gfn1 = _dot(gf, ng1)
    gfn2 = _dot(gf, ng2)

    onehot128 = jnp.pad(onehotn, ((0, 0), (0, H - G)))
    hist = _sc_hist(onehot128, src3, jnp.zeros((8, H), _f32))
    counts = jnp.sum(hist, axis=(0, 1))[:G]
    ends_v = jnp.cumsum(counts)
    starts_v = ends_v - counts
    starts8 = jnp.broadcast_to(
        jnp.zeros((32,), _f32).at[:G].set(starts_v), (8, 32))
    ends8 = jnp.broadcast_to(
        jnp.zeros((32,), _f32).at[:G].set(ends_v), (8, 32))

    x0, xi1, xj1 = _tc_node0(
        xpad, _pad_rows(wn01, 8), wn02, wn03,
        _bias_mat(bn01, bn02, bn03), w1i_1, w1j_1)

    eb1 = _bias_mat(ba1, ba2, ba3, be11, be12, be13)
    ga1a, gb1a = _sc_gather2(xi1, xj1, dst3a, src3a)
    ga1b, gb1b = _sc_gather2(xi1, xj1, dst3b, src3b)
    ef0a, msg1a = _tc_edge1(
        attr8[:EH], ga1a, gb1a, _pad_rows(wa1, 8), wa2, wa3, w1e_1, e12,
        e13, gfe1, starts8, ends8, eb1, 0)
    ef0b, msg1b = _tc_edge1(
        attr8[EH:], ga1b, gb1b, _pad_rows(wa1, 8), wa2, wa3, w1e_1, e12,
        e13, gfe1, starts8, ends8, eb1, EH)
    seg1a = _sc_scatter(msg1a, dst3a, zerosH)
    seg1b = _sc_scatter(msg1b, dst3b, zerosH)
    x1, xi2, xj2 = _tc_node1(
        x0, seg1a, seg1b, onehotn, nx1, na1, gfn1, n12, n13,
        _bias_mat(bn11, bn12, bn13), w1i_2, w1j_2)

    eb2 = _bias_mat(be21, be22, be23)
    ga2a, gb2a = _sc_gather2(xi2, xj2, dst3a, src3a)
    ga2b, gb2b = _sc_gather2(xi2, xj2, dst3b, src3b)
    msg2a = _tc_edge2(ef0a, msg1a, ga2a, gb2a, w1e_2, e22, e23, gfe2,
                      starts8, ends8, eb2, 0)
    msg2b = _tc_edge2(ef0b, msg1b, ga2b, gb2b, w1e_2, e22, e23, gfe2,
                      starts8, ends8, eb2, EH)
    seg2a = _sc_scatter(msg2a, dst3a, zerosH)
    seg2b = _sc_scatter(msg2b, dst3b, zerosH)

    wc3p = jnp.zeros((H, H), _f32).at[:, :3].set(wc3)
    bc3p = jnp.zeros((H,), _f32).at[:3].set(bc3)
    out = _tc_node2(
        x1, seg2a, seg2b, onehotn, nx2, na2, gfn2, n22, n23,
        _bias_mat(bn21, bn22, bn23), wc1, wc2, wc3p,
        _bias_mat(bc1, bc2, bc3p))
    return out[:, :3]
